# baseline (speedup 1.0000x reference)
import functools

import jax
import jax.numpy as jnp
from jax.experimental import pallas as pl
from jax.experimental.pallas import tpu as pltpu
from jax.experimental.pallas import tpu_sc as plsc

_NC, _NS = 2, 16
_NW = _NC * _NS

_WINDOW = 128
_NBUF = 4


def _sc_gather(src, row_idx, n_rows, d):
    window = _WINDOW
    assert n_rows % (_NW * window * _NBUF) == 0
    b_per_w = n_rows // _NW
    n_groups = b_per_w // (window * _NBUF)
    mesh = plsc.VectorSubcoreMesh(core_axis_name="c", subcore_axis_name="s")

    @functools.partial(
        pl.kernel,
        out_type=jax.ShapeDtypeStruct((n_rows, 1, d), src.dtype),
        mesh=mesh,
        scratch_types=[
            pltpu.VMEM((b_per_w,), jnp.int32),
            *([pltpu.VMEM((window, 1, d), src.dtype)] * _NBUF),
            *([pltpu.SemaphoreType.DMA] * _NBUF),
            pltpu.SemaphoreType.DMA,
        ],
    )
    def gather_kernel(tab_hbm, idx_hbm, out_hbm, idx_v, *bufs_sems):
        rows = bufs_sems[:_NBUF]
        sems = bufs_sems[_NBUF:2 * _NBUF]
        isem = bufs_sems[2 * _NBUF]
        wid = jax.lax.axis_index("s") * _NC + jax.lax.axis_index("c")
        base = wid * b_per_w
        pltpu.async_copy(idx_hbm.at[pl.ds(base, b_per_w)], idx_v, isem).wait()

        @pl.loop(0, n_groups)
        def _(t):
            g0 = t * (window * _NBUF)
            for b in range(_NBUF):
                off = g0 + b * window
                pltpu.async_copy(
                    tab_hbm.at[idx_v.at[pl.ds(off, window)]], rows[b], sems[b]
                )
            for b in range(_NBUF):
                off = g0 + b * window
                pltpu.make_async_copy(
                    tab_hbm.at[idx_v.at[pl.ds(off, window)]], rows[b], sems[b]
                ).wait()
                pltpu.sync_copy(rows[b], out_hbm.at[pl.ds(base + off, window)])

    return gather_kernel(src, row_idx)


def _pack_bf16_pairs(tables):
    F, V, D = tables.shape
    b = jax.lax.bitcast_convert_type(tables, jnp.uint32)
    r = (b + jnp.uint32(0x7FFF) + ((b >> 16) & jnp.uint32(1))) >> 16
    packed = r[..., 0::2] | (r[..., 1::2] << 16)
    return packed.reshape(F * V, D // 2)


def _make_mlp_body(n_fields):
    def _mlp_body(x_ref, w1_ref, b1_ref, w2_ref, b2_ref, w3_ref, b3_ref,
                  o_ref):
        top = jnp.uint32(0xFFFF0000)
        pieces = []
        for f in range(n_fields):
            xi = x_ref[f]
            lo = jax.lax.bitcast_convert_type(xi << 16, jnp.float32)
            hi = jax.lax.bitcast_convert_type(xi & top, jnp.float32)
            pieces.append(lo)
            pieces.append(hi)
        xm = jnp.concatenate(pieces, axis=1).astype(jnp.bfloat16)
        h = jnp.dot(xm, w1_ref[...], preferred_element_type=jnp.float32)
        h = jnp.maximum(h + b1_ref[...], 0.0).astype(jnp.bfloat16)
        h = jnp.dot(h, w2_ref[...], preferred_element_type=jnp.float32)
        h = jnp.maximum(h + b2_ref[...], 0.0).astype(jnp.bfloat16)
        z = jnp.dot(h, w3_ref[...], preferred_element_type=jnp.float32)
        o_ref[...] = jax.nn.sigmoid(z + b3_ref[...])

    return _mlp_body


def _tc_mlp(xfm, W1p, b1, W2, b2, W3, b3, block_b=512):
    n_fields, bsz, dw = xfm.shape
    hdim = W2.shape[0]
    assert bsz % block_b == 0
    grid = (bsz // block_b,)
    full = lambda shape: pl.BlockSpec(shape, lambda i: (0,) * len(shape))
    return pl.pallas_call(
        _make_mlp_body(n_fields),
        grid=grid,
        in_specs=[
            pl.BlockSpec((n_fields, block_b, dw), lambda i: (0, i, 0)),
            full((n_fields * 2 * dw, hdim)),
            full((1, hdim)),
            full((hdim, hdim)),
            full((1, hdim)),
            full((hdim, 1)),
            full((1, 1)),
        ],
        out_specs=pl.BlockSpec((block_b, 1), lambda i: (i, 0)),
        out_shape=jax.ShapeDtypeStruct((bsz, 1), jnp.float32),
    )(xfm, W1p, b1, W2, b2, W3, b3)


def kernel(indices, tables, W1, b1, W2, b2, W3, b3):
    B, F = indices.shape
    _, V, D = tables.shape
    H = W1.shape[1]
    offsets = (jnp.arange(F, dtype=jnp.int32) * V)[None, :]
    flat_idx = ((indices + offsets).T).reshape(B * F)
    packed = _pack_bf16_pairs(tables)
    x = _sc_gather(packed[:, None, :], flat_idx, B * F, D // 2)
    xfm = x.reshape(F, B, D // 2)
    W1p = (W1.reshape(F, D // 2, 2, H).transpose(0, 2, 1, 3)
           .reshape(F * D, H))
    return _tc_mlp(
        xfm,
        W1p.astype(jnp.bfloat16),
        b1.reshape(1, -1),
        W2.astype(jnp.bfloat16),
        b2.reshape(1, -1),
        W3.astype(jnp.bfloat16),
        b3.reshape(1, 1),
    )


# speedup vs baseline: 4.9429x; 4.9429x over previous
import functools

import jax
import jax.numpy as jnp
from jax.experimental import pallas as pl
from jax.experimental.pallas import tpu as pltpu
from jax.experimental.pallas import tpu_sc as plsc

_NC, _NS = 2, 16
_NW = _NC * _NS


_WINDOW = 128
_NBUF = 4


def _sc_gather(tables, row_idx, n_rows, n_flat, d):
    window = _WINDOW
    assert n_rows % (_NW * window * _NBUF) == 0
    b_per_w = n_rows // _NW
    n_groups = b_per_w // (window * _NBUF)
    mesh = plsc.VectorSubcoreMesh(core_axis_name="c", subcore_axis_name="s")

    @functools.partial(
        pl.kernel,
        out_type=jax.ShapeDtypeStruct((n_rows, 1, d), tables.dtype),
        mesh=mesh,
        scratch_types=[
            pltpu.VMEM((b_per_w,), jnp.int32),
            *([pltpu.VMEM((window, 1, d), tables.dtype)] * _NBUF),
            *([pltpu.SemaphoreType.DMA] * _NBUF),
            pltpu.SemaphoreType.DMA,
        ],
    )
    def gather_kernel(tab_hbm, idx_hbm, out_hbm, idx_v, *bufs_sems):
        rows = bufs_sems[:_NBUF]
        sems = bufs_sems[_NBUF:2 * _NBUF]
        isem = bufs_sems[2 * _NBUF]
        tab = tab_hbm
        wid = jax.lax.axis_index("s") * _NC + jax.lax.axis_index("c")
        base = wid * b_per_w
        pltpu.async_copy(idx_hbm.at[pl.ds(base, b_per_w)], idx_v, isem).wait()

        @pl.loop(0, n_groups)
        def _(t):
            g0 = t * (window * _NBUF)
            for b in range(_NBUF):
                off = g0 + b * window
                pltpu.async_copy(
                    tab.at[idx_v.at[pl.ds(off, window)]], rows[b], sems[b]
                )
            for b in range(_NBUF):
                off = g0 + b * window
                pltpu.make_async_copy(
                    tab.at[idx_v.at[pl.ds(off, window)]], rows[b], sems[b]
                ).wait()
                pltpu.sync_copy(rows[b], out_hbm.at[pl.ds(base + off, window)])

    return gather_kernel(tables, row_idx)


def _make_mlp_body(n_fields):
    def _mlp_body(x_ref, w1_ref, b1_ref, w2_ref, b2_ref, w3_ref, b3_ref,
                  o_ref):
        xm = jnp.concatenate(
            [x_ref[f] for f in range(n_fields)], axis=1
        ).astype(jnp.bfloat16)
        h = jnp.dot(xm, w1_ref[...], preferred_element_type=jnp.float32)
        h = jnp.maximum(h + b1_ref[...], 0.0).astype(jnp.bfloat16)
        h = jnp.dot(h, w2_ref[...], preferred_element_type=jnp.float32)
        h = jnp.maximum(h + b2_ref[...], 0.0).astype(jnp.bfloat16)
        z = jnp.dot(h, w3_ref[...], preferred_element_type=jnp.float32)
        o_ref[...] = jax.nn.sigmoid(z + b3_ref[...])

    return _mlp_body


def _tc_mlp(xfm, W1, b1, W2, b2, W3, b3, block_b=512):
    n_fields, bsz, d = xfm.shape
    hdim = W2.shape[0]
    assert bsz % block_b == 0
    grid = (bsz // block_b,)
    full = lambda shape: pl.BlockSpec(shape, lambda i: (0,) * len(shape))
    return pl.pallas_call(
        _make_mlp_body(n_fields),
        grid=grid,
        in_specs=[
            pl.BlockSpec((n_fields, block_b, d), lambda i: (0, i, 0)),
            full((n_fields * d, hdim)),
            full((1, hdim)),
            full((hdim, hdim)),
            full((1, hdim)),
            full((hdim, 1)),
            full((1, 1)),
        ],
        out_specs=pl.BlockSpec((block_b, 1), lambda i: (i, 0)),
        out_shape=jax.ShapeDtypeStruct((bsz, 1), jnp.float32),
    )(xfm, W1, b1, W2, b2, W3, b3)


def kernel(indices, tables, W1, b1, W2, b2, W3, b3):
    B, F = indices.shape
    _, V, D = tables.shape
    n_chunks = 2
    bc = B // n_chunks
    offsets = (jnp.arange(F, dtype=jnp.int32) * V)[None, :]
    src = tables[:, :, None, :].reshape(F * V, 1, D)
    w1 = W1.astype(jnp.bfloat16)
    w2 = W2.astype(jnp.bfloat16)
    w3 = W3.astype(jnp.bfloat16)
    b1r, b2r, b3r = b1.reshape(1, -1), b2.reshape(1, -1), b3.reshape(1, 1)
    outs = []
    for c in range(n_chunks):
        idx_c = indices[c * bc:(c + 1) * bc]
        flat_idx = ((idx_c + offsets).T).reshape(bc * F)
        x = _sc_gather(src, flat_idx, bc * F, F * V, D)
        xfm = x.reshape(F, bc, D)
        outs.append(_tc_mlp(xfm, w1, b1r, w2, b2r, w3, b3r))
    return jnp.concatenate(outs, axis=0)
